# baseline (speedup 1.0000x reference)
import functools

import jax
import jax.numpy as jnp
from jax import lax
from jax.experimental import pallas as pl
from jax.experimental.pallas import tpu as pltpu
from jax.experimental.pallas import tpu_sc as plsc

N = 10000
NE = 5000
P = 320000
D = 128

NC, NS, L = 2, 16, 16
NW = NC * NS
NEpad = 5120
Npad = 10240

K = 128
NCHW = 79
NCH = NW * NCHW
P2 = NCH * K
CS = 64
CH0, CH1 = 118, 40


def _agg_split(c, s):
    nch = jnp.where(c == 0, CH0, CH1)
    cb = jnp.where(c == 0, s * CH0, NS * CH0 + s * CH1)
    return nch, cb


def _mesh():
    return plsc.VectorSubcoreMesh(core_axis_name="c", subcore_axis_name="s")


def _proj_body(x_ref, w_ref, b_ref, o_ref):
    o_ref[...] = lax.dot_general(
        x_ref[...], w_ref[...], (((1,), (1,)), ((), ())),
        preferred_element_type=jnp.float32) + b_ref[...]


def _proj(feats, W, b):
    return pl.pallas_call(
        _proj_body,
        grid=(10,),
        in_specs=[
            pl.BlockSpec((1000, D), lambda i: (i, 0)),
            pl.BlockSpec((D, D), lambda i: (0, 0)),
            pl.BlockSpec((1, D), lambda i: (0, 0)),
        ],
        out_specs=pl.BlockSpec((1000, D), lambda i: (i, 0)),
        out_shape=jax.ShapeDtypeStruct((N, D), jnp.float32),
    )(feats, W, b.reshape(1, D))


def _den_body(idx, wts, zeros1, dep, dvp, se, sv, ib, wb, zb, semi, sems):
    c = lax.axis_index("c")
    s = lax.axis_index("s")
    wid = s * NC + c
    cb = wid * NCHW
    re, rv = NEpad // NS, Npad // NS

    pltpu.sync_copy(zeros1, zb)
    pltpu.sync_copy(zb.at[pl.ds(0, re)], se.at[pl.ds(s * re, re)])
    pltpu.sync_copy(zb, sv.at[pl.ds(s * rv, rv)])
    plsc.subcore_barrier()

    def start_in(i, b):
        pltpu.async_copy(idx.at[cb + i], ib.at[b], semi.at[b])
        pltpu.async_copy(wts.at[cb + i], wb.at[b], semi.at[b])

    def wait_in(b):
        pltpu.make_async_copy(idx.at[cb], ib.at[b], semi.at[b]).wait()
        pltpu.make_async_copy(wts.at[cb], wb.at[b], semi.at[b]).wait()

    def start_sc(b):
        pltpu.async_copy(wb.at[b, 0], se.at[ib.at[b, 0]], sems.at[b],
                         add=True)
        pltpu.async_copy(wb.at[b, 1], sv.at[ib.at[b, 1]], sems.at[b],
                         add=True)

    def wait_sc(b):
        pltpu.make_async_copy(wb.at[b, 0], se.at[ib.at[b, 0]],
                              sems.at[b]).wait()
        pltpu.make_async_copy(wb.at[b, 1], sv.at[ib.at[b, 1]],
                              sems.at[b]).wait()

    start_in(0, 0)

    def step(i, carry):
        b = i % 2
        nb = 1 - b
        wait_in(b)
        start_sc(b)

        @pl.when(i >= 1)
        def _():
            wait_sc(nb)

        @pl.when(i <= NCHW - 2)
        def _():
            start_in(i + 1, nb)

        return carry

    lax.fori_loop(0, NCHW, step, 0)
    wait_sc((NCHW - 1) % 2)

    plsc.subcore_barrier()
    pltpu.sync_copy(se.at[pl.ds(s * re, re)], zb.at[pl.ds(0, re)])
    pltpu.sync_copy(zb.at[pl.ds(0, re)], dep.at[pl.ds(c * NEpad + s * re, re)])
    pltpu.sync_copy(sv.at[pl.ds(s * rv, rv)], zb)
    pltpu.sync_copy(zb, dvp.at[pl.ds(c * Npad + s * rv, rv)])


def _sc_den(idx, wts, zeros1):
    f = pl.kernel(
        _den_body,
        out_type=(jax.ShapeDtypeStruct((NC * NEpad,), jnp.float32),
                  jax.ShapeDtypeStruct((NC * Npad,), jnp.float32)),
        mesh=_mesh(),
        scratch_types=[
            pltpu.VMEM_SHARED((NEpad,), jnp.float32),
            pltpu.VMEM_SHARED((Npad,), jnp.float32),
            pltpu.VMEM((2, 2, K), jnp.int32),
            pltpu.VMEM((2, 2, K), jnp.float32),
            pltpu.VMEM((Npad // NS,), jnp.float32),
            pltpu.SemaphoreType.DMA((2,)),
            pltpu.SemaphoreType.DMA((2,)),
        ],
    )
    return f(idx, wts, zeros1)


def _inv_body(d_ref, o_ref):
    o_ref[...] = 1.0 / jnp.maximum(d_ref[0] + d_ref[1], 1e-12)


def _inv(dp, rows):
    return pl.pallas_call(
        _inv_body,
        out_shape=jax.ShapeDtypeStruct((rows, D), jnp.float32),
    )(dp.reshape(NC, rows, D)).reshape(rows * D)


def _wts_body(idx, wts, inv_e, inv_v, alpha, beta,
              ib, wb, ge, gv, av, bv, semi, semg, semo):
    c = lax.axis_index("c")
    s = lax.axis_index("s")
    wid = s * NC + c
    cb = wid * NCHW

    def start_in(i, b):
        pltpu.async_copy(idx.at[cb + i], ib.at[b], semi.at[b])
        pltpu.async_copy(wts.at[cb + i], wb.at[b], semi.at[b])

    def wait_in(b):
        pltpu.make_async_copy(idx.at[cb], ib.at[b], semi.at[b]).wait()
        pltpu.make_async_copy(wts.at[cb], wb.at[b], semi.at[b]).wait()

    def start_g(b):
        pltpu.async_copy(inv_e.at[ib.at[b, 0]], ge.at[b], semg.at[b])
        pltpu.async_copy(inv_v.at[ib.at[b, 1]], gv.at[b], semg.at[b])

    def wait_g(b):
        pltpu.make_async_copy(inv_e.at[ib.at[b, 0]], ge.at[b],
                              semg.at[b]).wait()
        pltpu.make_async_copy(inv_v.at[ib.at[b, 1]], gv.at[b],
                              semg.at[b]).wait()

    def start_out(i, b):
        off = (cb + i) * K
        pltpu.async_copy(av.at[b], alpha.at[pl.ds(off, K)], semo.at[b])
        pltpu.async_copy(bv.at[b], beta.at[pl.ds(off, K)], semo.at[b])

    def wait_out(b):
        pltpu.make_async_copy(av.at[b], alpha.at[pl.ds(0, K)],
                              semo.at[b]).wait()
        pltpu.make_async_copy(bv.at[b], beta.at[pl.ds(0, K)],
                              semo.at[b]).wait()

    start_in(0, 0)
    wait_in(0)
    start_g(0)
    start_in(1, 1)

    def step(i, carry):
        b = i % 2
        nb = 1 - b
        wait_g(b)
        for j in range(K // L):
            sl = pl.ds(j * L, L)
            av[b, sl] = wb[b, 0, sl] * ge[b, sl]
            bv[b, sl] = wb[b, 1, sl] * gv[b, sl]

        @pl.when(i >= 1)
        def _():
            wait_out(nb)

        start_out(i, b)

        @pl.when(i <= NCHW - 2)
        def _():
            wait_in(nb)
            start_g(nb)

        @pl.when(i <= NCHW - 3)
        def _():
            start_in(i + 2, b)

        return carry

    lax.fori_loop(0, NCHW, step, 0)
    wait_out((NCHW - 1) % 2)


def _sc_wts(idx, wts, inv_e, inv_v):
    f = pl.kernel(
        _wts_body,
        out_type=(jax.ShapeDtypeStruct((P2,), jnp.float32),
                  jax.ShapeDtypeStruct((P2,), jnp.float32)),
        mesh=_mesh(),
        scratch_types=[
            pltpu.VMEM((2, 2, K), jnp.int32),
            pltpu.VMEM((2, 2, K), jnp.float32),
            pltpu.VMEM((2, K), jnp.float32),
            pltpu.VMEM((2, K), jnp.float32),
            pltpu.VMEM((2, K), jnp.float32),
            pltpu.VMEM((2, K), jnp.float32),
            pltpu.SemaphoreType.DMA((2,)),
            pltpu.SemaphoreType.DMA((2,)),
            pltpu.SemaphoreType.DMA((2,)),
        ],
    )
    return f(idx, wts, inv_e, inv_v)


def _agg_body(mpad, src, idx, wts, zeros2, part,
              acc, rows, ib, wb, sb, semi, semg, semd):
    c = lax.axis_index("c")
    s = lax.axis_index("s")
    nch, cb = _agg_split(c, s)
    rpt = mpad // NS

    pltpu.sync_copy(zeros2.at[pl.ds(0, CS)], rows.at[pl.ds(0, CS)])
    for k in range(rpt // CS):
        pltpu.sync_copy(rows.at[pl.ds(0, CS)],
                        acc.at[pl.ds(s * rpt + k * CS, CS)])
    plsc.subcore_barrier()

    def start_in(i, b):
        pltpu.async_copy(idx.at[cb + i], ib.at[b], semi.at[b])
        pltpu.async_copy(wts.at[cb + i], wb.at[b], semi.at[b])

    def wait_in(b):
        pltpu.make_async_copy(idx.at[cb], ib.at[b], semi.at[b]).wait()
        pltpu.make_async_copy(wts.at[cb], wb.at[b], semi.at[b]).wait()

    def start_g(b):
        pltpu.async_copy(src.at[ib.at[b, 0]], rows.at[pl.ds(b * K, K)],
                         semg.at[b])

    def wait_g(b):
        pltpu.make_async_copy(src.at[ib.at[b, 0]], rows.at[pl.ds(b * K, K)],
                              semg.at[b]).wait()

    def start_d(b):
        pltpu.async_copy(rows.at[pl.ds(b * K, K)], acc.at[sb.at[b]],
                         semd.at[b], add=True)

    def wait_d(b):
        pltpu.make_async_copy(rows.at[pl.ds(b * K, K)], acc.at[sb.at[b]],
                              semd.at[b]).wait()

    start_in(0, 0)
    wait_in(0)
    start_g(0)
    start_in(1, 1)

    def step(i, carry):
        b = i % 2
        nb = 1 - b
        wait_g(b)
        for j in range(K // L):
            sl = pl.ds(j * L, L)
            sb[b, sl] = ib[b, 1, sl]

        def kb_body(kb, carry2):
            w16 = wb[b, pl.ds(kb * L, L)]
            for r in range(L):
                ws = lax.broadcast_in_dim(
                    lax.slice(w16, (r,), (r + 1,)), (L,), (0,))
                row = b * K + kb * L + r
                for j in range(D // L):
                    sl = pl.ds(j * L, L)
                    rows[row, sl] = rows[row, sl] * ws
            return carry2

        lax.fori_loop(0, K // L, kb_body, 0)

        @pl.when(i >= 1)
        def _():
            wait_d(nb)

        start_d(b)

        @pl.when(i <= nch - 2)
        def _():
            wait_in(nb)
            start_g(nb)

        @pl.when(i <= nch - 3)
        def _():
            start_in(i + 2, b)

        return carry

    lax.fori_loop(0, nch, step, 0)
    wait_d((nch - 1) % 2)

    plsc.subcore_barrier()
    for k in range(rpt // CS):
        pltpu.sync_copy(acc.at[pl.ds(s * rpt + k * CS, CS)],
                        rows.at[pl.ds(0, CS)])
        pltpu.sync_copy(rows.at[pl.ds(0, CS)],
                        part.at[c, pl.ds(s * rpt + k * CS, CS)])


@functools.lru_cache(maxsize=None)
def _make_agg(src_rows, mpad):
    del src_rows
    return pl.kernel(
        functools.partial(_agg_body, mpad),
        out_type=jax.ShapeDtypeStruct((NC, mpad, D), jnp.float32),
        mesh=_mesh(),
        scratch_types=[
            pltpu.VMEM_SHARED((mpad, D), jnp.float32),
            pltpu.VMEM((2 * K, D), jnp.float32),
            pltpu.VMEM((2, 2, K), jnp.int32),
            pltpu.VMEM((2, K), jnp.float32),
            pltpu.VMEM((2, K), jnp.int32),
            pltpu.SemaphoreType.DMA((2,)),
            pltpu.SemaphoreType.DMA((2,)),
            pltpu.SemaphoreType.DMA((2,)),
        ],
    )


def _sc_agg(src, idx, wts, zeros2, mpad):
    return _make_agg(src.shape[0], mpad)(src, idx, wts, zeros2)


def _comb_body(p_ref, o_ref):
    o_ref[...] = p_ref[0] + p_ref[1]


def _comb(part, mpad):
    nb = mpad // 1024
    return pl.pallas_call(
        _comb_body,
        grid=(nb,),
        in_specs=[pl.BlockSpec((NC, 1024, D), lambda i: (0, i, 0))],
        out_specs=pl.BlockSpec((1024, D), lambda i: (i, 0)),
        out_shape=jax.ShapeDtypeStruct((mpad, D), jnp.float32),
    )(part)


def _smax_body(p_ref, o_ref):
    x = p_ref[0] + p_ref[1]
    m = jnp.max(x, axis=1, keepdims=True)
    e = jnp.exp(x - m)
    o_ref[...] = e / jnp.sum(e, axis=1, keepdims=True)


def _smax(part):
    return pl.pallas_call(
        _smax_body,
        grid=(10,),
        in_specs=[pl.BlockSpec((NC, 1000, D), lambda i: (0, i, 0))],
        out_specs=pl.BlockSpec((1000, D), lambda i: (i, 0)),
        out_shape=jax.ShapeDtypeStruct((N, D), jnp.float32),
    )(part)


def _pack2(a, b):
    return jnp.stack([a, b]).reshape(2, NCH, K).transpose(1, 0, 2)


def kernel(feats, pair_v, pair_e, v2e_weight, e2v_weight, W, b):
    pad = P2 - P
    pv2 = jnp.pad(pair_v, (0, pad))
    pe2 = jnp.pad(pair_e, (0, pad))
    w12 = jnp.pad(v2e_weight, (0, pad))
    w22 = jnp.pad(e2v_weight, (0, pad))
    idx_ev = _pack2(pe2, pv2)
    idx_ve = _pack2(pv2, pe2)
    wden = _pack2(w12, w22)
    zeros1 = jnp.zeros((Npad // NS,), jnp.float32)
    zeros2 = jnp.zeros((CS, D), jnp.float32)

    H0 = _proj(feats, W, b)

    dep, dvp = _sc_den(idx_ev, wden, zeros1)
    inv_e = _inv(dep, NEpad // D)
    inv_v = _inv(dvp, Npad // D)
    alpha, beta = _sc_wts(idx_ev, wden, inv_e, inv_v)
    wa = alpha.reshape(NCH, K)
    wb = beta.reshape(NCH, K)

    p = _sc_agg(H0, idx_ve, wa, zeros2, NEpad)
    Y1 = _comb(p, NEpad)
    p = _sc_agg(Y1, idx_ev, wb, zeros2, Npad)
    X1 = _comb(p, Npad)[:N]
    p = _sc_agg(X1, idx_ve, wa, zeros2, NEpad)
    Y2 = _comb(p, NEpad)
    p = _sc_agg(Y2, idx_ev, wb, zeros2, Npad)
    return _smax(p)


# speedup vs baseline: 1.0184x; 1.0184x over previous
import functools

import jax
import jax.numpy as jnp
from jax import lax
from jax.experimental import pallas as pl
from jax.experimental.pallas import tpu as pltpu
from jax.experimental.pallas import tpu_sc as plsc

N = 10000
NE = 5000
P = 320000
D = 128

NC, NS, L = 2, 16, 16
NW = NC * NS
NEpad = 5120
Npad = 10240

K = 128
NCHW = 79
NCH = NW * NCHW
P2 = NCH * K
CS = 64
CH0, CH1 = 112, 46


def _agg_split(c, s):
    nch = jnp.where(c == 0, CH0, CH1)
    cb = jnp.where(c == 0, s * CH0, NS * CH0 + s * CH1)
    return nch, cb


def _mesh():
    return plsc.VectorSubcoreMesh(core_axis_name="c", subcore_axis_name="s")


def _proj_body(x_ref, w_ref, b_ref, o_ref):
    o_ref[...] = lax.dot_general(
        x_ref[...], w_ref[...], (((1,), (1,)), ((), ())),
        preferred_element_type=jnp.float32) + b_ref[...]


def _proj(feats, W, b):
    return pl.pallas_call(
        _proj_body,
        grid=(10,),
        in_specs=[
            pl.BlockSpec((1000, D), lambda i: (i, 0)),
            pl.BlockSpec((D, D), lambda i: (0, 0)),
            pl.BlockSpec((1, D), lambda i: (0, 0)),
        ],
        out_specs=pl.BlockSpec((1000, D), lambda i: (i, 0)),
        out_shape=jax.ShapeDtypeStruct((N, D), jnp.float32),
    )(feats, W, b.reshape(1, D))


def _den_body(idx, wts, zeros1, dep, dvp, se, sv, ib, wb, zb, semi, sems):
    c = lax.axis_index("c")
    s = lax.axis_index("s")
    wid = s * NC + c
    cb = wid * NCHW
    re, rv = NEpad // NS, Npad // NS

    pltpu.sync_copy(zeros1, zb)
    pltpu.sync_copy(zb.at[pl.ds(0, re)], se.at[pl.ds(s * re, re)])
    pltpu.sync_copy(zb, sv.at[pl.ds(s * rv, rv)])
    plsc.subcore_barrier()

    def start_in(i, b):
        pltpu.async_copy(idx.at[cb + i], ib.at[b], semi.at[b])
        pltpu.async_copy(wts.at[cb + i], wb.at[b], semi.at[b])

    def wait_in(b):
        pltpu.make_async_copy(idx.at[cb], ib.at[b], semi.at[b]).wait()
        pltpu.make_async_copy(wts.at[cb], wb.at[b], semi.at[b]).wait()

    def start_sc(b):
        pltpu.async_copy(wb.at[b, 0], se.at[ib.at[b, 0]], sems.at[b],
                         add=True)
        pltpu.async_copy(wb.at[b, 1], sv.at[ib.at[b, 1]], sems.at[b],
                         add=True)

    def wait_sc(b):
        pltpu.make_async_copy(wb.at[b, 0], se.at[ib.at[b, 0]],
                              sems.at[b]).wait()
        pltpu.make_async_copy(wb.at[b, 1], sv.at[ib.at[b, 1]],
                              sems.at[b]).wait()

    start_in(0, 0)

    def step(i, carry):
        b = i % 2
        nb = 1 - b
        wait_in(b)
        start_sc(b)

        @pl.when(i >= 1)
        def _():
            wait_sc(nb)

        @pl.when(i <= NCHW - 2)
        def _():
            start_in(i + 1, nb)

        return carry

    lax.fori_loop(0, NCHW, step, 0)
    wait_sc((NCHW - 1) % 2)

    plsc.subcore_barrier()
    pltpu.sync_copy(se.at[pl.ds(s * re, re)], zb.at[pl.ds(0, re)])
    pltpu.sync_copy(zb.at[pl.ds(0, re)], dep.at[pl.ds(c * NEpad + s * re, re)])
    pltpu.sync_copy(sv.at[pl.ds(s * rv, rv)], zb)
    pltpu.sync_copy(zb, dvp.at[pl.ds(c * Npad + s * rv, rv)])


def _sc_den(idx, wts, zeros1):
    f = pl.kernel(
        _den_body,
        out_type=(jax.ShapeDtypeStruct((NC * NEpad,), jnp.float32),
                  jax.ShapeDtypeStruct((NC * Npad,), jnp.float32)),
        mesh=_mesh(),
        scratch_types=[
            pltpu.VMEM_SHARED((NEpad,), jnp.float32),
            pltpu.VMEM_SHARED((Npad,), jnp.float32),
            pltpu.VMEM((2, 2, K), jnp.int32),
            pltpu.VMEM((2, 2, K), jnp.float32),
            pltpu.VMEM((Npad // NS,), jnp.float32),
            pltpu.SemaphoreType.DMA((2,)),
            pltpu.SemaphoreType.DMA((2,)),
        ],
    )
    return f(idx, wts, zeros1)


def _inv_body(d_ref, o_ref):
    o_ref[...] = 1.0 / jnp.maximum(d_ref[0] + d_ref[1], 1e-12)


def _inv(dp, rows):
    return pl.pallas_call(
        _inv_body,
        out_shape=jax.ShapeDtypeStruct((rows, D), jnp.float32),
    )(dp.reshape(NC, rows, D)).reshape(rows * D)


def _wts_body(idx, wts, inv_e, inv_v, alpha, beta,
              ib, wb, ge, gv, av, bv, semi, semg, semo):
    c = lax.axis_index("c")
    s = lax.axis_index("s")
    wid = s * NC + c
    cb = wid * NCHW

    def start_in(i, b):
        pltpu.async_copy(idx.at[cb + i], ib.at[b], semi.at[b])
        pltpu.async_copy(wts.at[cb + i], wb.at[b], semi.at[b])

    def wait_in(b):
        pltpu.make_async_copy(idx.at[cb], ib.at[b], semi.at[b]).wait()
        pltpu.make_async_copy(wts.at[cb], wb.at[b], semi.at[b]).wait()

    def start_g(b):
        pltpu.async_copy(inv_e.at[ib.at[b, 0]], ge.at[b], semg.at[b])
        pltpu.async_copy(inv_v.at[ib.at[b, 1]], gv.at[b], semg.at[b])

    def wait_g(b):
        pltpu.make_async_copy(inv_e.at[ib.at[b, 0]], ge.at[b],
                              semg.at[b]).wait()
        pltpu.make_async_copy(inv_v.at[ib.at[b, 1]], gv.at[b],
                              semg.at[b]).wait()

    def start_out(i, b):
        off = (cb + i) * K
        pltpu.async_copy(av.at[b], alpha.at[pl.ds(off, K)], semo.at[b])
        pltpu.async_copy(bv.at[b], beta.at[pl.ds(off, K)], semo.at[b])

    def wait_out(b):
        pltpu.make_async_copy(av.at[b], alpha.at[pl.ds(0, K)],
                              semo.at[b]).wait()
        pltpu.make_async_copy(bv.at[b], beta.at[pl.ds(0, K)],
                              semo.at[b]).wait()

    start_in(0, 0)
    wait_in(0)
    start_g(0)
    start_in(1, 1)

    def step(i, carry):
        b = i % 2
        nb = 1 - b
        wait_g(b)
        for j in range(K // L):
            sl = pl.ds(j * L, L)
            av[b, sl] = wb[b, 0, sl] * ge[b, sl]
            bv[b, sl] = wb[b, 1, sl] * gv[b, sl]

        @pl.when(i >= 1)
        def _():
            wait_out(nb)

        start_out(i, b)

        @pl.when(i <= NCHW - 2)
        def _():
            wait_in(nb)
            start_g(nb)

        @pl.when(i <= NCHW - 3)
        def _():
            start_in(i + 2, b)

        return carry

    lax.fori_loop(0, NCHW, step, 0)
    wait_out((NCHW - 1) % 2)


def _sc_wts(idx, wts, inv_e, inv_v):
    f = pl.kernel(
        _wts_body,
        out_type=(jax.ShapeDtypeStruct((P2,), jnp.float32),
                  jax.ShapeDtypeStruct((P2,), jnp.float32)),
        mesh=_mesh(),
        scratch_types=[
            pltpu.VMEM((2, 2, K), jnp.int32),
            pltpu.VMEM((2, 2, K), jnp.float32),
            pltpu.VMEM((2, K), jnp.float32),
            pltpu.VMEM((2, K), jnp.float32),
            pltpu.VMEM((2, K), jnp.float32),
            pltpu.VMEM((2, K), jnp.float32),
            pltpu.SemaphoreType.DMA((2,)),
            pltpu.SemaphoreType.DMA((2,)),
            pltpu.SemaphoreType.DMA((2,)),
        ],
    )
    return f(idx, wts, inv_e, inv_v)


def _agg_body(mpad, src, idx, wts, zeros2, part,
              acc, rows, ib, wb, sb, semi, semg, semd):
    c = lax.axis_index("c")
    s = lax.axis_index("s")
    nch, cb = _agg_split(c, s)
    rpt = mpad // NS

    pltpu.sync_copy(zeros2.at[pl.ds(0, CS)], rows.at[pl.ds(0, CS)])
    for k in range(rpt // CS):
        pltpu.sync_copy(rows.at[pl.ds(0, CS)],
                        acc.at[pl.ds(s * rpt + k * CS, CS)])
    plsc.subcore_barrier()

    def start_in(i, b):
        pltpu.async_copy(idx.at[cb + i], ib.at[b], semi.at[b])
        pltpu.async_copy(wts.at[cb + i], wb.at[b], semi.at[b])

    def wait_in(b):
        pltpu.make_async_copy(idx.at[cb], ib.at[b], semi.at[b]).wait()
        pltpu.make_async_copy(wts.at[cb], wb.at[b], semi.at[b]).wait()

    def start_g(b):
        pltpu.async_copy(src.at[ib.at[b, 0]], rows.at[pl.ds(b * K, K)],
                         semg.at[b])

    def wait_g(b):
        pltpu.make_async_copy(src.at[ib.at[b, 0]], rows.at[pl.ds(b * K, K)],
                              semg.at[b]).wait()

    def start_d(b):
        pltpu.async_copy(rows.at[pl.ds(b * K, K)], acc.at[sb.at[b]],
                         semd.at[b], add=True)

    def wait_d(b):
        pltpu.make_async_copy(rows.at[pl.ds(b * K, K)], acc.at[sb.at[b]],
                              semd.at[b]).wait()

    start_in(0, 0)
    wait_in(0)
    start_g(0)
    start_in(1, 1)

    def step(i, carry):
        b = i % 2
        nb = 1 - b
        wait_g(b)
        for j in range(K // L):
            sl = pl.ds(j * L, L)
            sb[b, sl] = ib[b, 1, sl]

        def kb_body(kb, carry2):
            w16 = wb[b, pl.ds(kb * L, L)]
            for r in range(L):
                ws = lax.broadcast_in_dim(
                    lax.slice(w16, (r,), (r + 1,)), (L,), (0,))
                row = b * K + kb * L + r
                for j in range(D // L):
                    sl = pl.ds(j * L, L)
                    rows[row, sl] = rows[row, sl] * ws
            return carry2

        lax.fori_loop(0, K // L, kb_body, 0)

        @pl.when(i >= 1)
        def _():
            wait_d(nb)

        start_d(b)

        @pl.when(i <= nch - 2)
        def _():
            wait_in(nb)
            start_g(nb)

        @pl.when(i <= nch - 3)
        def _():
            start_in(i + 2, b)

        return carry

    lax.fori_loop(0, nch, step, 0)
    wait_d((nch - 1) % 2)

    plsc.subcore_barrier()
    for k in range(rpt // CS):
        pltpu.sync_copy(acc.at[pl.ds(s * rpt + k * CS, CS)],
                        rows.at[pl.ds(0, CS)])
        pltpu.sync_copy(rows.at[pl.ds(0, CS)],
                        part.at[c, pl.ds(s * rpt + k * CS, CS)])


@functools.lru_cache(maxsize=None)
def _make_agg(src_rows, mpad):
    del src_rows
    return pl.kernel(
        functools.partial(_agg_body, mpad),
        out_type=jax.ShapeDtypeStruct((NC, mpad, D), jnp.float32),
        mesh=_mesh(),
        scratch_types=[
            pltpu.VMEM_SHARED((mpad, D), jnp.float32),
            pltpu.VMEM((2 * K, D), jnp.float32),
            pltpu.VMEM((2, 2, K), jnp.int32),
            pltpu.VMEM((2, K), jnp.float32),
            pltpu.VMEM((2, K), jnp.int32),
            pltpu.SemaphoreType.DMA((2,)),
            pltpu.SemaphoreType.DMA((2,)),
            pltpu.SemaphoreType.DMA((2,)),
        ],
    )


def _sc_agg(src, idx, wts, zeros2, mpad):
    return _make_agg(src.shape[0], mpad)(src, idx, wts, zeros2)


def _comb_body(p_ref, o_ref):
    o_ref[...] = p_ref[0] + p_ref[1]


def _comb(part, mpad):
    nb = mpad // 1024
    return pl.pallas_call(
        _comb_body,
        grid=(nb,),
        in_specs=[pl.BlockSpec((NC, 1024, D), lambda i: (0, i, 0))],
        out_specs=pl.BlockSpec((1024, D), lambda i: (i, 0)),
        out_shape=jax.ShapeDtypeStruct((mpad, D), jnp.float32),
    )(part)


def _smax_body(p_ref, o_ref):
    x = p_ref[0] + p_ref[1]
    m = jnp.max(x, axis=1, keepdims=True)
    e = jnp.exp(x - m)
    o_ref[...] = e / jnp.sum(e, axis=1, keepdims=True)


def _smax(part):
    return pl.pallas_call(
        _smax_body,
        grid=(10,),
        in_specs=[pl.BlockSpec((NC, 1000, D), lambda i: (0, i, 0))],
        out_specs=pl.BlockSpec((1000, D), lambda i: (i, 0)),
        out_shape=jax.ShapeDtypeStruct((N, D), jnp.float32),
    )(part)


def _pack2(a, b):
    return jnp.stack([a, b]).reshape(2, NCH, K).transpose(1, 0, 2)


def kernel(feats, pair_v, pair_e, v2e_weight, e2v_weight, W, b):
    pad = P2 - P
    pv2 = jnp.pad(pair_v, (0, pad))
    pe2 = jnp.pad(pair_e, (0, pad))
    w12 = jnp.pad(v2e_weight, (0, pad))
    w22 = jnp.pad(e2v_weight, (0, pad))
    idx_ev = _pack2(pe2, pv2)
    idx_ve = _pack2(pv2, pe2)
    wden = _pack2(w12, w22)
    zeros1 = jnp.zeros((Npad // NS,), jnp.float32)
    zeros2 = jnp.zeros((CS, D), jnp.float32)

    H0 = _proj(feats, W, b)

    dep, dvp = _sc_den(idx_ev, wden, zeros1)
    inv_e = _inv(dep, NEpad // D)
    inv_v = _inv(dvp, Npad // D)
    alpha, beta = _sc_wts(idx_ev, wden, inv_e, inv_v)
    wa = alpha.reshape(NCH, K)
    wb = beta.reshape(NCH, K)

    p = _sc_agg(H0, idx_ve, wa, zeros2, NEpad)
    Y1 = _comb(p, NEpad)
    p = _sc_agg(Y1, idx_ev, wb, zeros2, Npad)
    X1 = _comb(p, Npad)[:N]
    p = _sc_agg(X1, idx_ve, wa, zeros2, NEpad)
    Y2 = _comb(p, NEpad)
    p = _sc_agg(Y2, idx_ev, wb, zeros2, Npad)
    return _smax(p)
